# baseline (speedup 1.0000x reference)
import functools

import jax
import jax.numpy as jnp
from jax import lax
from jax.experimental import pallas as pl
from jax.experimental.pallas import tpu as pltpu
from jax.experimental.pallas import tpu_sc as plsc

N = 10000
E = 320000
C = 128
DE = 16
FF = 512

NC = 2
NS = 16
NW = NC * NS
L = 16

CK = 80
E_PAD = 327680
EPT = E_PAD // NW
NCHUNK = EPT // CK
SCK = 8
NSC = NCHUNK // SCK
N_PAD = 10240
NPT = N_PAD // NS

NBK = 32
CAPB = 640
CK2 = 128
NCH2 = CAPB // CK2
NLOC = N_PAD // NBK

NB = 400
NGRID = N // NB
EB = 8000
EGRID = E // EB

_f32 = jnp.float32
_i32 = jnp.int32


def _pre_body(x_ref, w_ref, as_ref, ad_ref, h_ref, asrc_ref, adst_ref):
    h = jnp.dot(x_ref[...], w_ref[...], preferred_element_type=_f32)
    h_ref[...] = h
    asrc_ref[...] = jnp.sum(h * as_ref[...], axis=1, keepdims=True)
    adst_ref[...] = jnp.sum(h * ad_ref[...], axis=1, keepdims=True)


def _pre(x, W, att_src_row, att_dst_row):
    return pl.pallas_call(
        _pre_body,
        grid=(NGRID,),
        in_specs=[
            pl.BlockSpec((NB, C), lambda i: (i, 0)),
            pl.BlockSpec((C, C), lambda i: (0, 0)),
            pl.BlockSpec((1, C), lambda i: (0, 0)),
            pl.BlockSpec((1, C), lambda i: (0, 0)),
        ],
        out_specs=[
            pl.BlockSpec((NB, C), lambda i: (i, 0)),
            pl.BlockSpec((NB, 1), lambda i: (i, 0)),
            pl.BlockSpec((NB, 1), lambda i: (i, 0)),
        ],
        out_shape=[
            jax.ShapeDtypeStruct((N, C), _f32),
            jax.ShapeDtypeStruct((N, 1), _f32),
            jax.ShapeDtypeStruct((N, 1), _f32),
        ],
    )(x, W, att_src_row, att_dst_row)


def _edge_body(ea_ref, lew_ref, ae_ref, aedge_ref):
    we = jnp.sum(lew_ref[...] * ae_ref[...], axis=1, keepdims=True)
    aedge_ref[...] = jnp.dot(ea_ref[...], we, preferred_element_type=_f32)


def _edge(edge_attr, lin_edge_W, att_edge_row):
    return pl.pallas_call(
        _edge_body,
        grid=(EGRID,),
        in_specs=[
            pl.BlockSpec((EB, DE), lambda i: (i, 0)),
            pl.BlockSpec((DE, C), lambda i: (0, 0)),
            pl.BlockSpec((1, C), lambda i: (0, 0)),
        ],
        out_specs=pl.BlockSpec((EB, 1), lambda i: (i, 0)),
        out_shape=jax.ShapeDtypeStruct((E, 1), _f32),
    )(edge_attr, lin_edge_W, att_edge_row)


@functools.partial(
    pl.kernel,
    out_type=(
        jax.ShapeDtypeStruct((NW, NBK * CAPB), _i32),
        jax.ShapeDtypeStruct((NW, NBK * CAPB), _i32),
        jax.ShapeDtypeStruct((NW, NBK * CAPB), _f32),
        jax.ShapeDtypeStruct((NW, NBK), _i32),
    ),
    mesh=plsc.VectorSubcoreMesh(core_axis_name="c", subcore_axis_name="s"),
    compiler_params=pltpu.CompilerParams(needs_layout_passes=False),
    scratch_types=(
        pltpu.VMEM((N,), _f32),
        pltpu.VMEM((N,), _f32),
        pltpu.VMEM((SCK * 3, CK), _i32),
        pltpu.VMEM((SCK * 3, CK), _i32),
        pltpu.VMEM((NBK * CAPB,), _i32),
        pltpu.VMEM((NBK * CAPB,), _i32),
        pltpu.VMEM((NBK * CAPB,), _f32),
        pltpu.VMEM((NBK,), _i32),
        pltpu.SemaphoreType.DMA,
        pltpu.SemaphoreType.DMA,
        pltpu.SemaphoreType.DMA,
    ),
)
def _sc_bin(packed_hbm, asrc_hbm, adst_hbm,
            ssrc_hbm, sdst_hbm, sex_hbm, cnt_hbm,
            asrc_v, adst_v, pkb_a, pkb_b, bsrc_v, bdst_v, bex_v,
            cnt_v, sem_pa, sem_pb, sem_f):
    cid = lax.axis_index("c")
    sid = lax.axis_index("s")
    wid = sid * NC + cid

    pltpu.sync_copy(asrc_hbm, asrc_v)
    pltpu.sync_copy(adst_hbm, adst_v)
    pltpu.sync_copy(packed_hbm.at[wid, 0], pkb_a)

    io16 = lax.broadcasted_iota(_i32, (L,), 0)

    def _chunk(k, pkb_cur, cnts):
        cnt_lo, cnt_hi = cnts
        for g in range(CK // L):
            s_idx = pkb_cur[3 * k, pl.ds(g * L, L)]
            d_idx = pkb_cur[3 * k + 1, pl.ds(g * L, L)]
            ab = plsc.bitcast(pkb_cur[3 * k + 2, pl.ds(g * L, L)], _f32)
            a = (plsc.load_gather(asrc_v, [s_idx])
                 + plsc.load_gather(adst_v, [d_idx]) + ab)
            a = jnp.where(a > 0.0, a, 0.2 * a)
            ex = jnp.exp(a)
            bkt = jnp.bitwise_and(jnp.right_shift(d_idx, 5), NBK - 1)
            live = ex > 0.0
            for b in range(NBK):
                m = jnp.logical_and(bkt == b, live)
                mi = m.astype(_i32)
                csum = plsc.cumsum(mi)
                cv = cnt_lo if b < L else cnt_hi
                off = cv[b % L]
                pos = (b * CAPB) + off + csum - mi
                plsc.store_scatter(bsrc_v, [pos], s_idx, mask=m)
                plsc.store_scatter(bdst_v, [pos], d_idx, mask=m)
                plsc.store_scatter(bex_v, [pos], ex, mask=m)
                pc = csum[L - 1]
                upd = jnp.where(io16 == (b % L), pc, 0)
                if b < L:
                    cnt_lo = cnt_lo + upd
                else:
                    cnt_hi = cnt_hi + upd
        return (cnt_lo, cnt_hi)

    def _super(si, pkb_cur, pkb_nxt, sem_pk_nxt, cnts):
        sn = jnp.minimum(si + 1, NSC - 1)
        pltpu.async_copy(packed_hbm.at[wid, sn], pkb_nxt, sem_pk_nxt)
        cnts = lax.fori_loop(
            0, SCK, lambda k, cn: _chunk(k, pkb_cur, cn), cnts)
        pltpu.make_async_copy(packed_hbm.at[wid, sn], pkb_nxt, sem_pk_nxt).wait()
        return cnts

    def pair_body(q, cnts):
        cnts = _super(2 * q, pkb_a, pkb_b, sem_pb, cnts)
        cnts = _super(2 * q + 1, pkb_b, pkb_a, sem_pa, cnts)
        return cnts

    cnt_lo, cnt_hi = lax.fori_loop(
        0, NSC // 2, pair_body,
        (jnp.zeros((L,), _i32), jnp.zeros((L,), _i32)))

    zi = jnp.zeros((L,), _i32)
    zf = jnp.zeros((L,), _f32)
    for b in range(NBK):
        cv = cnt_lo if b < L else cnt_hi
        nb = cv[b % L]
        for t in range(CK2 // L):
            bsrc_v[pl.ds(b * CAPB + nb + t * L, L)] = zi
            bdst_v[pl.ds(b * CAPB + nb + t * L, L)] = zi
            bex_v[pl.ds(b * CAPB + nb + t * L, L)] = zf
    cnt_v[pl.ds(0, L)] = cnt_lo
    cnt_v[pl.ds(L, L)] = cnt_hi

    hf = [pltpu.async_copy(bsrc_v, ssrc_hbm.at[wid], sem_f),
          pltpu.async_copy(bdst_v, sdst_hbm.at[wid], sem_f),
          pltpu.async_copy(bex_v, sex_hbm.at[wid], sem_f)]
    for hh in hf:
        hh.wait()
    pltpu.sync_copy(cnt_v, cnt_hbm.at[wid])


@functools.partial(
    pl.kernel,
    out_type=(
        jax.ShapeDtypeStruct((N_PAD, C), _f32),
        jax.ShapeDtypeStruct((N_PAD, L), _f32),
    ),
    mesh=plsc.VectorSubcoreMesh(core_axis_name="c", subcore_axis_name="s"),
    compiler_params=pltpu.CompilerParams(needs_layout_passes=False),
    scratch_types=(
        pltpu.VMEM((NLOC, C), _f32),
        pltpu.VMEM((NLOC, L), _f32),
        pltpu.VMEM((CAPB,), _i32),
        pltpu.VMEM((CAPB,), _i32),
        pltpu.VMEM((CAPB,), _f32),
        pltpu.VMEM((CAPB,), _i32),
        pltpu.VMEM((CAPB,), _i32),
        pltpu.VMEM((CAPB,), _f32),
        pltpu.VMEM((CK2, C), _f32),
        pltpu.VMEM((CK2, C), _f32),
        pltpu.VMEM((CK2,), _i32),
        pltpu.VMEM((CK2,), _i32),
        pltpu.VMEM((NBK,), _i32),
        pltpu.SMEM((NBK,), _i32),
        pltpu.SMEM((CK2,), _i32),
        pltpu.SemaphoreType.DMA,
        pltpu.SemaphoreType.DMA,
        pltpu.SemaphoreType.DMA,
        pltpu.SemaphoreType.DMA,
    ),
)
def _sc_agg(h_hbm, ssrc_hbm, sdst_hbm, sex_hbm, cntt_hbm, z3_hbm, z4_hbm,
            out_hbm, den_hbm,
            acc_v, dac_v, csa, cda, cea, csb, cdb, ceb, rows0, rows1,
            idx0, idx1, cnt_v, cnt_sm, dl_sm, sem_ca, sem_cb, sem_g0, sem_g1):
    cid = lax.axis_index("c")
    sid = lax.axis_index("s")
    wid = sid * NC + cid

    pltpu.sync_copy(z3_hbm, acc_v)
    pltpu.sync_copy(z4_hbm, dac_v)
    pltpu.sync_copy(cntt_hbm.at[wid], cnt_v)
    oh0 = jnp.where(lax.broadcasted_iota(_i32, (L,), 0) == 0,
                    jnp.float32(1.0), jnp.float32(0.0))
    for g in range(NBK // L):
        cvals = cnt_v[pl.ds(g * L, L)]
        for l in range(L):
            cnt_sm[g * L + l] = cvals[l]

    rows = (rows0, rows1)
    sems_g = (sem_g0, sem_g1)
    idxs = (idx0, idx1)

    def _process(p, cs, cd, ce):
        n = cnt_sm[p]
        hs = [None, None]

        @pl.when(n > 0)
        def _():
            for g in range(CK2 // L):
                idxs[0][pl.ds(g * L, L)] = cs[pl.ds(g * L, L)]
            hs[0] = pltpu.async_copy(
                h_hbm.at[idxs[0]], rows[0], sems_g[0])

        for c in range(NCH2):
            rc = c % 2

            if c + 1 < NCH2:
                @pl.when((c + 1) * CK2 < n)
                def _():
                    for g in range(CK2 // L):
                        idxs[1 - rc][pl.ds(g * L, L)] = (
                            cs[pl.ds((c + 1) * CK2 + g * L, L)])
                    hs[1 - rc] = pltpu.async_copy(
                        h_hbm.at[idxs[1 - rc]],
                        rows[1 - rc], sems_g[1 - rc])

            @pl.when(c * CK2 < n)
            def _():
                for g in range(CK2 // L):
                    d16 = cd[pl.ds(c * CK2 + g * L, L)]
                    dl16 = jnp.bitwise_or(
                        jnp.left_shift(jnp.right_shift(d16, 10), 5),
                        jnp.bitwise_and(d16, NBK - 1))
                    for l in range(L):
                        dl_sm[g * L + l] = dl16[l]
                hs[rc].wait()

                def edge_body(e, carry):
                    dl = dl_sm[e]
                    wv = plsc.load_gather(
                        ce, [jnp.full((L,), c * CK2 + e, _i32)])
                    dac_v[dl, pl.ds(0, L)] = (
                        dac_v[dl, pl.ds(0, L)] + wv * oh0)
                    for j in range(C // L):
                        acc_v[dl, pl.ds(j * L, L)] = (
                            acc_v[dl, pl.ds(j * L, L)]
                            + rows[rc][e, pl.ds(j * L, L)] * wv)
                    return carry

                lax.fori_loop(0, CK2, edge_body, 0)

    def pair_body(t, carry):
        p0 = 2 * t
        cell = pl.ds(wid * CAPB, CAPB)
        pltpu.sync_copy(ssrc_hbm.at[p0, cell], csa)
        pltpu.sync_copy(sdst_hbm.at[p0, cell], cda)
        pltpu.sync_copy(sex_hbm.at[p0, cell], cea)
        hb = [pltpu.async_copy(ssrc_hbm.at[p0 + 1, cell], csb, sem_cb),
              pltpu.async_copy(sdst_hbm.at[p0 + 1, cell], cdb, sem_cb),
              pltpu.async_copy(sex_hbm.at[p0 + 1, cell], ceb, sem_cb)]
        _process(p0, csa, cda, cea)
        for h in hb:
            h.wait()
        _process(p0 + 1, csb, cdb, ceb)
        return carry

    lax.fori_loop(0, NW // 2, pair_body, 0)

    for q in range(N_PAD // (NBK * NBK)):
        pltpu.sync_copy(acc_v.at[pl.ds(q * NBK, NBK)],
                        out_hbm.at[pl.ds(q * NBK * NBK + wid * NBK, NBK)])
        pltpu.sync_copy(dac_v.at[pl.ds(q * NBK, NBK)],
                        den_hbm.at[pl.ds(q * NBK * NBK + wid * NBK, NBK)])


def _ln(v, g, b):
    m = jnp.mean(v, axis=1, keepdims=True)
    d = v - m
    var = jnp.mean(d * d, axis=1, keepdims=True)
    return d * jax.lax.rsqrt(var + 1e-5) * g + b


def _post_body(p_ref, d2_ref, x_ref, b_ref,
               w1_ref, b1_ref, w2_ref, b2_ref,
               g1_ref, be1_ref, g2_ref, be2_ref, y_ref):
    denom = jnp.sum(d2_ref[...], axis=1, keepdims=True) + 1e-16
    agg = p_ref[...] / denom + b_ref[...]
    v = _ln(agg + x_ref[...], g1_ref[...], be1_ref[...])
    ff = jnp.maximum(
        jnp.dot(v, w1_ref[...], preferred_element_type=_f32) + b1_ref[...], 0.0)
    ffo = jnp.dot(ff, w2_ref[...], preferred_element_type=_f32) + b2_ref[...]
    y_ref[...] = _ln(v + ffo, g2_ref[...], be2_ref[...])


def _post(p, d2, x, bias_row, ff_W1, b1_row, ff_W2, b2_row,
          g1_row, be1_row, g2_row, be2_row):
    row = lambda i: (0, 0)
    return pl.pallas_call(
        _post_body,
        grid=(NGRID,),
        in_specs=[
            pl.BlockSpec((NB, C), lambda i: (i, 0)),
            pl.BlockSpec((NB, L), lambda i: (i, 0)),
            pl.BlockSpec((NB, C), lambda i: (i, 0)),
            pl.BlockSpec((1, C), row),
            pl.BlockSpec((C, FF), row),
            pl.BlockSpec((1, FF), row),
            pl.BlockSpec((FF, C), row),
            pl.BlockSpec((1, C), row),
            pl.BlockSpec((1, C), row),
            pl.BlockSpec((1, C), row),
            pl.BlockSpec((1, C), row),
            pl.BlockSpec((1, C), row),
        ],
        out_specs=pl.BlockSpec((NB, C), lambda i: (i, 0)),
        out_shape=jax.ShapeDtypeStruct((N, C), _f32),
    )(p, d2, x, bias_row, ff_W1, b1_row, ff_W2, b2_row,
      g1_row, be1_row, g2_row, be2_row)


def kernel(x, edge_index, edge_attr, W, att_src, att_dst, lin_edge_W,
           att_edge, bias, ff_W1, ff_b1, ff_W2, ff_b2,
           ln1_g, ln1_b, ln2_g, ln2_b):
    src = edge_index[0]
    dst = edge_index[1]

    h, a_src, a_dst = _pre(x, W, att_src.reshape(1, C), att_dst.reshape(1, C))
    a_edge = _edge(edge_attr, lin_edge_W, att_edge.reshape(1, C))

    pad = E_PAD - E
    src_p = jnp.concatenate([src, jnp.zeros((pad,), _i32)])
    dst_p = jnp.concatenate([dst, jnp.zeros((pad,), _i32)])
    ae_p = jnp.concatenate([a_edge.reshape(E), jnp.full((pad,), -1e30, _f32)])
    ae_bits = lax.bitcast_convert_type(ae_p, _i32)
    packed = jnp.stack(
        [src_p.reshape(NW, NSC, SCK, CK), dst_p.reshape(NW, NSC, SCK, CK),
         ae_bits.reshape(NW, NSC, SCK, CK)],
        axis=3).reshape(NW, NSC, SCK * 3, CK)

    z3 = jnp.zeros((NLOC, C), _f32)
    z4 = jnp.zeros((NLOC, L), _f32)

    ssrc, sdst, sex, cnts = _sc_bin(packed, a_src.reshape(N), a_dst.reshape(N))
    out, den2 = _sc_agg(h, ssrc, sdst, sex, cnts.T, z3, z4)

    return _post(out, den2,
                 x, bias.reshape(1, C), ff_W1, ff_b1.reshape(1, FF),
                 ff_W2, ff_b2.reshape(1, C), ln1_g.reshape(1, C),
                 ln1_b.reshape(1, C), ln2_g.reshape(1, C), ln2_b.reshape(1, C))


# speedup vs baseline: 3.4591x; 3.4591x over previous
import functools

import jax
import jax.numpy as jnp
from jax import lax
from jax.experimental import pallas as pl
from jax.experimental.pallas import tpu as pltpu
from jax.experimental.pallas import tpu_sc as plsc

N = 10000
E = 320000
C = 128
DE = 16
FF = 512

NC = 2
NS = 16
NW = NC * NS
L = 16

CK = 80
E_PAD = 327680
EPT = E_PAD // NW
NCHUNK = EPT // CK
SCK = 8
NSC = NCHUNK // SCK
N_PAD = 10240
NPT = N_PAD // NS

NB = 400
NGRID = N // NB
EB = 8000
EGRID = E // EB

_f32 = jnp.float32


def _pre_body(x_ref, w_ref, as_ref, ad_ref, h_ref, asrc_ref, adst_ref):
    h = jnp.dot(x_ref[...], w_ref[...], preferred_element_type=_f32)
    h_ref[...] = h
    asrc_ref[...] = jnp.sum(h * as_ref[...], axis=1, keepdims=True)
    adst_ref[...] = jnp.sum(h * ad_ref[...], axis=1, keepdims=True)


def _pre(x, W, att_src_row, att_dst_row):
    return pl.pallas_call(
        _pre_body,
        grid=(NGRID,),
        in_specs=[
            pl.BlockSpec((NB, C), lambda i: (i, 0)),
            pl.BlockSpec((C, C), lambda i: (0, 0)),
            pl.BlockSpec((1, C), lambda i: (0, 0)),
            pl.BlockSpec((1, C), lambda i: (0, 0)),
        ],
        out_specs=[
            pl.BlockSpec((NB, C), lambda i: (i, 0)),
            pl.BlockSpec((NB, 1), lambda i: (i, 0)),
            pl.BlockSpec((NB, 1), lambda i: (i, 0)),
        ],
        out_shape=[
            jax.ShapeDtypeStruct((N, C), _f32),
            jax.ShapeDtypeStruct((N, 1), _f32),
            jax.ShapeDtypeStruct((N, 1), _f32),
        ],
    )(x, W, att_src_row, att_dst_row)


def _edge_body(ea_ref, lew_ref, ae_ref, aedge_ref):
    we = jnp.sum(lew_ref[...] * ae_ref[...], axis=1, keepdims=True)
    aedge_ref[...] = jnp.dot(ea_ref[...], we, preferred_element_type=_f32)


def _edge(edge_attr, lin_edge_W, att_edge_row):
    return pl.pallas_call(
        _edge_body,
        grid=(EGRID,),
        in_specs=[
            pl.BlockSpec((EB, DE), lambda i: (i, 0)),
            pl.BlockSpec((DE, C), lambda i: (0, 0)),
            pl.BlockSpec((1, C), lambda i: (0, 0)),
        ],
        out_specs=pl.BlockSpec((EB, 1), lambda i: (i, 0)),
        out_shape=jax.ShapeDtypeStruct((E, 1), _f32),
    )(edge_attr, lin_edge_W, att_edge_row)


@functools.partial(
    pl.kernel,
    out_type=(
        jax.ShapeDtypeStruct((N_PAD,), _f32),
        jax.ShapeDtypeStruct((N_PAD,), _f32),
        jax.ShapeDtypeStruct((N_PAD, C), _f32),
        jax.ShapeDtypeStruct((N_PAD, C), _f32),
    ),
    mesh=plsc.VectorSubcoreMesh(core_axis_name="c", subcore_axis_name="s"),
    compiler_params=pltpu.CompilerParams(needs_layout_passes=False),
    scratch_types=(
        pltpu.VMEM((N,), _f32),
        pltpu.VMEM((N,), _f32),
        pltpu.VMEM((3, CK), jnp.int32),
        pltpu.VMEM((3, CK), jnp.int32),
        pltpu.VMEM((CK,), _f32),
        pltpu.VMEM((CK, C), _f32),
        pltpu.VMEM((CK, C), _f32),
        pltpu.VMEM_SHARED((N_PAD,), _f32),
        pltpu.VMEM_SHARED((N_PAD, C), _f32),
        pltpu.SemaphoreType.DMA,
        pltpu.SemaphoreType.DMA,
        pltpu.SemaphoreType.DMA,
        pltpu.SemaphoreType.DMA,
    ),
)
def _sc_gat(h_hbm, packed_hbm, asrc_hbm, adst_hbm, z1_hbm, z2_hbm,
            den0_hbm, den1_hbm, out0_hbm, out1_hbm,
            asrc_v, adst_v, pk_a, pk_b, ex_v, rows_a, rows_b,
            den_sh, out_sh, sem_ga, sem_gb, sem_pa, sem_pb):
    cid = lax.axis_index("c")
    sid = lax.axis_index("s")
    wid = sid * NC + cid

    pltpu.sync_copy(z1_hbm, den_sh.at[pl.ds(sid * NPT, NPT)])
    pltpu.sync_copy(z2_hbm, out_sh.at[pl.ds(sid * NPT, NPT)])
    pltpu.sync_copy(asrc_hbm, asrc_v)
    pltpu.sync_copy(adst_hbm, adst_v)

    pltpu.sync_copy(packed_hbm.at[wid, 0], pk_a)
    pltpu.async_copy(h_hbm.at[pk_a.at[0]], rows_a, sem_ga)
    plsc.subcore_barrier()

    def _half(c, pk_cur, rows_cur, sem_cur, pk_nxt, rows_nxt, sem_nxt):
        cn = jnp.minimum(c + 1, NCHUNK - 1)
        pltpu.sync_copy(packed_hbm.at[wid, cn], pk_nxt)
        pltpu.async_copy(h_hbm.at[pk_nxt.at[0]], rows_nxt, sem_nxt)

        for g in range(CK // L):
            s_idx = pk_cur[0, pl.ds(g * L, L)]
            d_idx = pk_cur[1, pl.ds(g * L, L)]
            ab = plsc.bitcast(pk_cur[2, pl.ds(g * L, L)], _f32)
            a = (plsc.load_gather(asrc_v, [s_idx])
                 + plsc.load_gather(adst_v, [d_idx]) + ab)
            a = jnp.where(a > 0.0, a, 0.2 * a)
            ex_v[pl.ds(g * L, L)] = jnp.exp(a)
        pltpu.sync_copy(ex_v, den_sh.at[pk_cur.at[1]], add=True)

        pltpu.make_async_copy(h_hbm.at[pk_cur.at[0]], rows_cur, sem_cur).wait()

        @plsc.parallel_loop(0, CK, 1, unroll=4)
        def _scale(e):
            w = plsc.load_gather(ex_v, [jnp.full((L,), e, jnp.int32)])
            for j in range(C // L):
                rows_cur[e, pl.ds(j * L, L)] = rows_cur[e, pl.ds(j * L, L)] * w

        pltpu.sync_copy(rows_cur, out_sh.at[pk_cur.at[1]], add=True)

    def pair_body(p, carry):
        _half(2 * p, pk_a, rows_a, sem_ga, pk_b, rows_b, sem_gb)
        _half(2 * p + 1, pk_b, rows_b, sem_gb, pk_a, rows_a, sem_ga)
        return carry

    lax.fori_loop(0, NCHUNK // 2, pair_body, 0)
    pltpu.make_async_copy(h_hbm.at[pk_a.at[0]], rows_a, sem_ga).wait()
    plsc.subcore_barrier()

    sl = pl.ds(sid * NPT, NPT)

    @pl.when(cid == 0)
    def _():
        pltpu.sync_copy(den_sh.at[sl], den0_hbm.at[sl])
        pltpu.sync_copy(out_sh.at[sl], out0_hbm.at[sl])

    @pl.when(cid == 1)
    def _():
        pltpu.sync_copy(den_sh.at[sl], den1_hbm.at[sl])
        pltpu.sync_copy(out_sh.at[sl], out1_hbm.at[sl])


def _ln(v, g, b):
    m = jnp.mean(v, axis=1, keepdims=True)
    d = v - m
    var = jnp.mean(d * d, axis=1, keepdims=True)
    return d * jax.lax.rsqrt(var + 1e-5) * g + b


def _post_body(p0_ref, p1_ref, d0_ref, d1_ref, x_ref, b_ref,
               w1_ref, b1_ref, w2_ref, b2_ref,
               g1_ref, be1_ref, g2_ref, be2_ref, y_ref):
    denom = d0_ref[...] + d1_ref[...] + 1e-16
    agg = (p0_ref[...] + p1_ref[...]) / denom + b_ref[...]
    v = _ln(agg + x_ref[...], g1_ref[...], be1_ref[...])
    ff = jnp.maximum(
        jnp.dot(v, w1_ref[...], preferred_element_type=_f32) + b1_ref[...], 0.0)
    ffo = jnp.dot(ff, w2_ref[...], preferred_element_type=_f32) + b2_ref[...]
    y_ref[...] = _ln(v + ffo, g2_ref[...], be2_ref[...])


def _post(p0, p1, d0, d1, x, bias_row, ff_W1, b1_row, ff_W2, b2_row,
          g1_row, be1_row, g2_row, be2_row):
    row = lambda i: (0, 0)
    return pl.pallas_call(
        _post_body,
        grid=(NGRID,),
        in_specs=[
            pl.BlockSpec((NB, C), lambda i: (i, 0)),
            pl.BlockSpec((NB, C), lambda i: (i, 0)),
            pl.BlockSpec((NB, 1), lambda i: (i, 0)),
            pl.BlockSpec((NB, 1), lambda i: (i, 0)),
            pl.BlockSpec((NB, C), lambda i: (i, 0)),
            pl.BlockSpec((1, C), row),
            pl.BlockSpec((C, FF), row),
            pl.BlockSpec((1, FF), row),
            pl.BlockSpec((FF, C), row),
            pl.BlockSpec((1, C), row),
            pl.BlockSpec((1, C), row),
            pl.BlockSpec((1, C), row),
            pl.BlockSpec((1, C), row),
            pl.BlockSpec((1, C), row),
        ],
        out_specs=pl.BlockSpec((NB, C), lambda i: (i, 0)),
        out_shape=jax.ShapeDtypeStruct((N, C), _f32),
    )(p0, p1, d0, d1, x, bias_row, ff_W1, b1_row, ff_W2, b2_row,
      g1_row, be1_row, g2_row, be2_row)


def kernel(x, edge_index, edge_attr, W, att_src, att_dst, lin_edge_W,
           att_edge, bias, ff_W1, ff_b1, ff_W2, ff_b2,
           ln1_g, ln1_b, ln2_g, ln2_b):
    src = edge_index[0]
    dst = edge_index[1]

    h, a_src, a_dst = _pre(x, W, att_src.reshape(1, C), att_dst.reshape(1, C))
    a_edge = _edge(edge_attr, lin_edge_W, att_edge.reshape(1, C))

    pad = E_PAD - E
    src_p = jnp.concatenate([src, jnp.zeros((pad,), jnp.int32)])
    dst_p = jnp.concatenate([dst, jnp.zeros((pad,), jnp.int32)])
    ae_p = jnp.concatenate([a_edge.reshape(E), jnp.full((pad,), -1e30, _f32)])
    ae_bits = lax.bitcast_convert_type(ae_p, jnp.int32)
    packed = jnp.stack(
        [src_p.reshape(NW, NCHUNK, CK), dst_p.reshape(NW, NCHUNK, CK),
         ae_bits.reshape(NW, NCHUNK, CK)], axis=2)

    z1 = jnp.zeros((NPT,), _f32)
    z2 = jnp.zeros((NPT, C), _f32)

    den0, den1, out0, out1 = _sc_gat(h, packed, a_src.reshape(N),
                                     a_dst.reshape(N), z1, z2)

    return _post(out0, out1, den0.reshape(N_PAD, 1), den1.reshape(N_PAD, 1),
                 x, bias.reshape(1, C), ff_W1, ff_b1.reshape(1, FF),
                 ff_W2, ff_b2.reshape(1, C), ln1_g.reshape(1, C),
                 ln1_b.reshape(1, C), ln2_g.reshape(1, C), ln2_b.reshape(1, C))
